# baseline (speedup 1.0000x reference)
import functools

import jax
import jax.numpy as jnp
from jax import lax
from jax.experimental import pallas as pl
from jax.experimental.pallas import tpu as pltpu
from jax.experimental.pallas import tpu_sc as plsc

EMBED_DIM = 128
NUM_WORKERS = 32
CHUNK = 200
NBUF = 2


def kernel(input, table):
    batch, hist = input.shape
    n = batch * hist
    per_w = n // NUM_WORKERS
    n_chunks = per_w // CHUNK
    n_outer = n_chunks // NBUF

    idx = input.astype(jnp.int32)

    mesh = plsc.VectorSubcoreMesh(core_axis_name="c", subcore_axis_name="s")

    @functools.partial(
        pl.kernel,
        mesh=mesh,
        out_type=jax.ShapeDtypeStruct((batch, hist, EMBED_DIM), jnp.float32),
        scratch_types=[pltpu.VMEM((CHUNK,), jnp.int32)] * NBUF
        + [pltpu.VMEM((CHUNK, EMBED_DIM), jnp.float32)] * NBUF
        + [pltpu.SemaphoreType.DMA] * (2 * NBUF),
    )
    def gather_kernel(idx_hbm, table_hbm, out_hbm, *scratch):
        idx_v = scratch[:NBUF]
        rows_v = scratch[NBUF : 2 * NBUF]
        gsem = scratch[2 * NBUF : 3 * NBUF]
        ssem = scratch[3 * NBUF :]
        wid = lax.axis_index("s") * 2 + lax.axis_index("c")
        base = wid * per_w
        row0 = wid * (per_w // CHUNK)

        def start_gather(c, b):
            pltpu.sync_copy(idx_hbm.at[row0 + c], idx_v[b])
            pltpu.async_copy(table_hbm.at[idx_v[b]], rows_v[b], gsem[b])

        for b in range(NBUF):
            start_gather(b, b)

        def body(j, carry):
            for b in range(NBUF):
                c = j * NBUF + b
                pltpu.make_async_copy(
                    table_hbm.at[idx_v[b]], rows_v[b], gsem[b]
                ).wait()
                pltpu.async_copy(
                    rows_v[b], out_hbm.at[row0 + c], ssem[b]
                )
            for b in range(NBUF):
                c = j * NBUF + b

                @pl.when(j < n_outer - 1)
                def _():
                    pltpu.make_async_copy(
                        rows_v[b], out_hbm.at[row0 + c], ssem[b]
                    ).wait()
                    start_gather(c + NBUF, b)

            return carry

        lax.fori_loop(0, n_outer, body, 0)

        for b in range(NBUF):
            pltpu.make_async_copy(
                rows_v[b], out_hbm.at[row0], ssem[b]
            ).wait()

    return gather_kernel(idx, table)


# speedup vs baseline: 1.0301x; 1.0301x over previous
import functools

import jax
import jax.numpy as jnp
from jax import lax
from jax.experimental import pallas as pl
from jax.experimental.pallas import tpu as pltpu
from jax.experimental.pallas import tpu_sc as plsc

EMBED_DIM = 128
NUM_WORKERS = 32
CHUNK = 200
NBUF = 4


def kernel(input, table):
    batch, hist = input.shape
    n = batch * hist
    per_w = n // NUM_WORKERS
    n_chunks = per_w // CHUNK
    n_outer = n_chunks // NBUF

    idx = input.astype(jnp.int32)

    mesh = plsc.VectorSubcoreMesh(core_axis_name="c", subcore_axis_name="s")

    @functools.partial(
        pl.kernel,
        mesh=mesh,
        out_type=jax.ShapeDtypeStruct((batch, hist, EMBED_DIM), jnp.float32),
        scratch_types=[pltpu.VMEM((CHUNK,), jnp.int32)] * NBUF
        + [pltpu.VMEM((CHUNK, EMBED_DIM), jnp.float32)] * NBUF
        + [pltpu.SemaphoreType.DMA] * (2 * NBUF),
    )
    def gather_kernel(idx_hbm, table_hbm, out_hbm, *scratch):
        idx_v = scratch[:NBUF]
        rows_v = scratch[NBUF : 2 * NBUF]
        gsem = scratch[2 * NBUF : 3 * NBUF]
        ssem = scratch[3 * NBUF :]
        wid = lax.axis_index("s") * 2 + lax.axis_index("c")
        base = wid * per_w
        row0 = wid * (per_w // CHUNK)

        def start_gather(c, b):
            pltpu.sync_copy(idx_hbm.at[row0 + c], idx_v[b])
            pltpu.async_copy(table_hbm.at[idx_v[b]], rows_v[b], gsem[b])

        for b in range(NBUF):
            start_gather(b, b)

        def body(j, carry):
            for b in range(NBUF):
                c = j * NBUF + b
                pltpu.make_async_copy(
                    table_hbm.at[idx_v[b]], rows_v[b], gsem[b]
                ).wait()
                pltpu.async_copy(
                    rows_v[b], out_hbm.at[row0 + c], ssem[b]
                )
            for b in range(NBUF):
                c = j * NBUF + b

                @pl.when(j < n_outer - 1)
                def _():
                    pltpu.make_async_copy(
                        rows_v[b], out_hbm.at[row0 + c], ssem[b]
                    ).wait()
                    start_gather(c + NBUF, b)

            return carry

        lax.fori_loop(0, n_outer, body, 0)

        for b in range(NBUF):
            pltpu.make_async_copy(
                rows_v[b], out_hbm.at[row0], ssem[b]
            ).wait()

    return gather_kernel(idx, table)
